# baseline (speedup 1.0000x reference)
import functools

import jax
import jax.numpy as jnp
from jax import lax
from jax.experimental import pallas as pl
from jax.experimental.pallas import tpu as pltpu

B, N, DIM = 2, 8192, 1024
LIGHT_HEADS, LIGHT_DH, WINDOW = 8, 64, 64
HEAVY_HEADS, HEAVY_DH = 8, 64
NUM_HEAVY_Q, NUM_HEAVY_KV = 1024, 2048
N_ITERS, EPS, FETCH_K_RATIO = 50, 1.0, 9.0 / 8.0

ROWS_PER_BLK = 512
NB = N // ROWS_PER_BLK
WIN_PER_BLK = ROWS_PER_BLK // WINDOW
NWIN = N // WINDOW
NEG_MAX = -3.4028235e38

_P = jax.lax.Precision.DEFAULT


def _dot(a, b, dims):
    return lax.dot_general(a, b, (dims, ((), ())), precision=_P,
                           preferred_element_type=jnp.float32)


def _k1_body(off_ref, idx_ref, xc_ref, xp_ref, xn_ref, lng_ref, lnb_ref,
             wqkv_ref, wout_ref, nullq_ref, ro_ref, y_ref):
    b = pl.program_id(0)
    i = pl.program_id(1)
    xc = xc_ref[0]
    xp = xp_ref[0]
    xn = xn_ref[0]

    xfull = jnp.concatenate([xp, xc, xn], axis=0)
    mu = jnp.mean(xfull, axis=-1, keepdims=True)
    var = jnp.mean((xfull - mu) ** 2, axis=-1, keepdims=True)
    xl = (xfull - mu) / jnp.sqrt(var + 1e-5) * lng_ref[0] + lnb_ref[0]

    qkv = _dot(xl, wqkv_ref[...], ((1,), (1,)))

    wg = i * WIN_PER_BLK + lax.broadcasted_iota(
        jnp.int32, (WIN_PER_BLK, 1, 1), 0)
    maskp = wg == 0
    maskn = wg == NWIN - 1

    W = WIN_PER_BLK
    dl = LIGHT_HEADS * LIGHT_DH
    scale = LIGHT_DH ** -0.5

    def bdot(a_, b_, dims):
        return lax.dot_general(a_, b_, (dims, ((0,), (0,))), precision=_P,
                               preferred_element_type=jnp.float32)

    outs = []
    for h in range(LIGHT_HEADS):
        qh = qkv[WINDOW:WINDOW + ROWS_PER_BLK,
                 h * LIGHT_DH:(h + 1) * LIGHT_DH].reshape(W, WINDOW, LIGHT_DH)
        kh = qkv[:, dl + h * LIGHT_DH:dl + (h + 1) * LIGHT_DH]
        vh = qkv[:, 2 * dl + h * LIGHT_DH:2 * dl + (h + 1) * LIGHT_DH]
        kp = kh[:ROWS_PER_BLK].reshape(W, WINDOW, LIGHT_DH)
        km = kh[WINDOW:WINDOW + ROWS_PER_BLK].reshape(W, WINDOW, LIGHT_DH)
        kn = kh[2 * WINDOW:].reshape(W, WINDOW, LIGHT_DH)
        sp = bdot(qh, kp, ((2,), (2,))) * scale
        sm = bdot(qh, km, ((2,), (2,))) * scale
        sn = bdot(qh, kn, ((2,), (2,))) * scale
        sp = jnp.where(maskp, NEG_MAX, sp)
        sn = jnp.where(maskn, NEG_MAX, sn)
        sim = jnp.concatenate([sp, sm, sn], axis=-1)
        m = jnp.max(sim, axis=-1, keepdims=True)
        p = jnp.exp(sim - m)
        denom = jnp.sum(p, axis=-1, keepdims=True)
        vp = vh[:ROWS_PER_BLK].reshape(W, WINDOW, LIGHT_DH)
        vm = vh[WINDOW:WINDOW + ROWS_PER_BLK].reshape(W, WINDOW, LIGHT_DH)
        vn = vh[2 * WINDOW:].reshape(W, WINDOW, LIGHT_DH)
        o = (bdot(p[:, :, :WINDOW], vp, ((2,), (1,)))
             + bdot(p[:, :, WINDOW:2 * WINDOW], vm, ((2,), (1,)))
             + bdot(p[:, :, 2 * WINDOW:], vn, ((2,), (1,)))) / denom
        outs.append(o.reshape(ROWS_PER_BLK, LIGHT_DH))
    attnout = jnp.concatenate(outs, axis=1)

    y = _dot(attnout, wout_ref[...], ((1,), (1,)))
    y_ref[0] = y + nullq_ref[...]

    lo = off_ref[b * (NB + 1) + i]
    hi = off_ref[b * (NB + 1) + i + 1]

    def add_row(j, carry):
        p = idx_ref[b * NUM_HEAVY_Q + j] - i * ROWS_PER_BLK
        y_ref[0, pl.ds(p, 1), :] += ro_ref[0, pl.ds(j, 1), :]
        return carry

    lax.fori_loop(lo, hi, add_row, 0)


def _light(x, ln_g, ln_b, wqkv, wout, nullq, offsets, idx_q, ro):
    grid = (B, NB)
    return pl.pallas_call(
        _k1_body,
        grid_spec=pltpu.PrefetchScalarGridSpec(
            num_scalar_prefetch=2,
            grid=grid,
            in_specs=[
                pl.BlockSpec((1, ROWS_PER_BLK, DIM), lambda b, i, *_: (b, i, 0)),
                pl.BlockSpec((1, WINDOW, DIM),
                             lambda b, i, *_: (b, jnp.maximum(i * WIN_PER_BLK - 1, 0), 0)),
                pl.BlockSpec((1, WINDOW, DIM),
                             lambda b, i, *_: (b, jnp.minimum(i * WIN_PER_BLK + WIN_PER_BLK,
                                                              NWIN - 1), 0)),
                pl.BlockSpec((1, DIM), lambda b, i, *_: (0, 0)),
                pl.BlockSpec((1, DIM), lambda b, i, *_: (0, 0)),
                pl.BlockSpec((3 * 512, DIM), lambda b, i, *_: (0, 0)),
                pl.BlockSpec((DIM, 512), lambda b, i, *_: (0, 0)),
                pl.BlockSpec((1, DIM), lambda b, i, *_: (0, 0)),
                pl.BlockSpec((1, NUM_HEAVY_Q, DIM), lambda b, i, *_: (b, 0, 0)),
            ],
            out_specs=pl.BlockSpec((1, ROWS_PER_BLK, DIM),
                                   lambda b, i, *_: (b, i, 0)),
        ),
        out_shape=jax.ShapeDtypeStruct((B, N, DIM), jnp.float32),
    )(offsets.reshape(-1), idx_q.reshape(-1),
      x, x, x, ln_g, ln_b, wqkv, wout, nullq, ro)


def _k2_body(s_ref, logk_ref, mask_ref):
    s = s_ref[...]
    logk = logk_ref[:, 0:1]

    def it(_, carry):
        a, bb = carry
        sb = (s + bb) / EPS
        m = jnp.max(sb, axis=-1, keepdims=True)
        lse = jnp.log(jnp.sum(jnp.exp(sb - m), axis=-1, keepdims=True)) + m
        a = EPS * (logk - lse)
        bb = -jnp.maximum(s + a, 0.0)
        return a, bb

    a0 = jnp.zeros_like(s[:, 0:1])
    a, bb = lax.fori_loop(0, N_ITERS, it, (a0, -s))
    mask_ref[...] = jnp.exp((s + a + bb) / EPS)


def _coor_descent(s4, logk4):
    return pl.pallas_call(
        _k2_body,
        out_shape=jax.ShapeDtypeStruct((4, N), jnp.float32),
    )(s4, logk4)


def _k3_body(rq_ref, rkv_ref, g_ref, qw_ref, kvw_ref, nkv_ref, outwt_ref,
             nullq_ref, ro_ref, xn_ref, cn_ref):
    h = pl.program_id(1)
    g = g_ref[0]

    def rms(t):
        n = jnp.sqrt(jnp.sum(t * t, axis=-1, keepdims=True))
        return t / jnp.maximum(n, 1e-12) * (DIM ** 0.5) * g

    @pl.when(h == 0)
    def _():
        xn_ref[...] = rms(rq_ref[0])
        cn_ref[...] = rms(rkv_ref[0])

    xn = xn_ref[...]
    cn = cn_ref[...]

    q = _dot(xn, qw_ref[...], ((1,), (1,)))
    kvh = _dot(cn, kvw_ref[...], ((1,), (1,)))
    k = kvh[:, :HEAVY_DH]
    v = kvh[:, HEAVY_DH:]
    nk = nkv_ref[0, 0]
    nv = nkv_ref[1, 0]

    scale = HEAVY_DH ** -0.5
    sim = _dot(q, k, ((1,), (1,))) * scale
    sim_null = _dot(q, nk, ((1,), (1,))) * scale
    m = jnp.maximum(jnp.max(sim, axis=-1, keepdims=True), sim_null)
    p = jnp.exp(sim - m)
    p_null = jnp.exp(sim_null - m)
    denom = jnp.sum(p, axis=-1, keepdims=True) + p_null
    o = (_dot(p, v, ((1,), (0,))) + p_null * nv) / denom

    contrib = _dot(o, outwt_ref[...], ((1,), (0,)))

    @pl.when(h == 0)
    def _():
        ro_ref[0] = contrib - nullq_ref[...]

    @pl.when(h > 0)
    def _():
        ro_ref[0] = ro_ref[0] + contrib


def _heavy(rq, rkv, g, q_w, kv_w, null_kv4, out_wt, nullq):
    grid = (B, HEAVY_HEADS)
    return pl.pallas_call(
        _k3_body,
        grid=grid,
        in_specs=[
            pl.BlockSpec((1, NUM_HEAVY_Q, DIM), lambda b, h: (b, 0, 0)),
            pl.BlockSpec((1, NUM_HEAVY_KV, DIM), lambda b, h: (b, 0, 0)),
            pl.BlockSpec((1, DIM), lambda b, h: (0, 0)),
            pl.BlockSpec((HEAVY_DH, DIM), lambda b, h: (h, 0)),
            pl.BlockSpec((2 * HEAVY_DH, DIM), lambda b, h: (h, 0)),
            pl.BlockSpec((2, 1, 1, HEAVY_DH), lambda b, h: (0, h, 0, 0)),
            pl.BlockSpec((HEAVY_DH, DIM), lambda b, h: (h, 0)),
            pl.BlockSpec((1, DIM), lambda b, h: (0, 0)),
        ],
        out_specs=pl.BlockSpec((1, NUM_HEAVY_Q, DIM), lambda b, h: (b, 0, 0)),
        out_shape=jax.ShapeDtypeStruct((B, NUM_HEAVY_Q, DIM), jnp.float32),
        scratch_shapes=[
            pltpu.VMEM((NUM_HEAVY_Q, DIM), jnp.float32),
            pltpu.VMEM((NUM_HEAVY_KV, DIM), jnp.float32),
        ],
        compiler_params=pltpu.CompilerParams(
            dimension_semantics=("arbitrary", "arbitrary")),
    )(rq, rkv, g, q_w, kv_w, null_kv4, out_wt, nullq)


def kernel(x, ln_g, ln_b, light_qkv_w, light_out_w, q_route_tok, kv_route_tok,
           heavy_norm_g, null_kv, heavy_q_w, heavy_kv_w, heavy_out_w,
           null_q_token):
    nullq = null_q_token.reshape(1, DIM)

    rtoks = jnp.concatenate([q_route_tok, kv_route_tok], axis=0)
    s2 = jnp.einsum('bnd,rd->brn', x, rtoks)
    s4 = jnp.concatenate([s2[:, 0], s2[:, 1]], axis=0)

    kq = jnp.float32(min(NUM_HEAVY_Q * FETCH_K_RATIO, float(N)))
    kkv = jnp.float32(min(NUM_HEAVY_KV * FETCH_K_RATIO, float(N)))
    logk4 = jnp.log(jnp.maximum(
        jnp.stack([kq, kq, kkv, kkv])[:, None], 1e-20))
    logk4 = jnp.broadcast_to(logk4, (4, 128))

    scores = _coor_descent(s4, logk4)
    _, idx_q = lax.top_k(scores[:B], NUM_HEAVY_Q)
    _, idx_kv = lax.top_k(scores[B:], NUM_HEAVY_KV)
    idx_q = jnp.sort(idx_q, axis=-1)
    idx_kv = jnp.sort(idx_kv, axis=-1)

    rq = jnp.take_along_axis(x, idx_q[:, :, None], axis=1)
    rkv = jnp.take_along_axis(x, idx_kv[:, :, None], axis=1)

    null_kv4 = null_kv.reshape(2, HEAVY_HEADS, 1, HEAVY_DH)
    ro = _heavy(rq, rkv, heavy_norm_g.reshape(1, DIM), heavy_q_w, heavy_kv_w,
                null_kv4, heavy_out_w.T, nullq)

    bounds = jnp.arange(NB + 1, dtype=jnp.int32) * ROWS_PER_BLK
    offsets = jax.vmap(lambda r: jnp.searchsorted(r, bounds))(idx_q)
    offsets = offsets.astype(jnp.int32)

    return _light(x, ln_g.reshape(1, DIM), ln_b.reshape(1, DIM),
                  light_qkv_w, light_out_w, nullq, offsets, idx_q, ro)


# speedup vs baseline: 1.0801x; 1.0801x over previous
import functools

import jax
import jax.numpy as jnp
from jax import lax
from jax.experimental import pallas as pl
from jax.experimental.pallas import tpu as pltpu
from jax.experimental.pallas import tpu_sc as plsc

B, N, DIM = 2, 8192, 1024
LIGHT_HEADS, LIGHT_DH, WINDOW = 8, 64, 64
HEAVY_HEADS, HEAVY_DH = 8, 64
NUM_HEAVY_Q, NUM_HEAVY_KV = 1024, 2048
N_ITERS, EPS, FETCH_K_RATIO = 50, 1.0, 9.0 / 8.0

ROWS_PER_BLK = 512
NB = N // ROWS_PER_BLK
WIN_PER_BLK = ROWS_PER_BLK // WINDOW
NWIN = N // WINDOW
NEG_MAX = -3.4028235e38

_P = jax.lax.Precision.DEFAULT


def _dot(a, b, dims):
    return lax.dot_general(a, b, (dims, ((), ())), precision=_P,
                           preferred_element_type=jnp.float32)


def _k1_body(off_ref, idx_ref, xc_ref, xp_ref, xn_ref, lng_ref, lnb_ref,
             wqkv_ref, wout_ref, nullq_ref, ro_ref, y_ref):
    b = pl.program_id(0)
    i = pl.program_id(1)
    xc = xc_ref[0]
    xp = xp_ref[0]
    xn = xn_ref[0]

    xfull = jnp.concatenate([xp, xc, xn], axis=0)
    mu = jnp.mean(xfull, axis=-1, keepdims=True)
    var = jnp.mean((xfull - mu) ** 2, axis=-1, keepdims=True)
    xl = (xfull - mu) / jnp.sqrt(var + 1e-5) * lng_ref[0] + lnb_ref[0]

    qkv = _dot(xl, wqkv_ref[...], ((1,), (1,)))

    r = lax.broadcasted_iota(jnp.int32, (ROWS_PER_BLK, 640), 0)
    c = lax.broadcasted_iota(jnp.int32, (ROWS_PER_BLK, 640), 1)
    rel = c // WINDOW - r // WINDOW
    g = i * WIN_PER_BLK + c // WINDOW - 1
    valid = (rel >= 0) & (rel <= 2) & (g >= 0) & (g < NWIN)

    dl = LIGHT_HEADS * LIGHT_DH
    outs = []
    for h in range(LIGHT_HEADS):
        qh = qkv[WINDOW:WINDOW + ROWS_PER_BLK, h * LIGHT_DH:(h + 1) * LIGHT_DH]
        kh = qkv[:, dl + h * LIGHT_DH:dl + (h + 1) * LIGHT_DH]
        vh = qkv[:, 2 * dl + h * LIGHT_DH:2 * dl + (h + 1) * LIGHT_DH]
        sim = _dot(qh, kh, ((1,), (1,))) * (LIGHT_DH ** -0.5)
        sim = jnp.where(valid, sim, NEG_MAX)
        m = jnp.max(sim, axis=-1, keepdims=True)
        p = jnp.exp(sim - m)
        attn = p / jnp.sum(p, axis=-1, keepdims=True)
        outs.append(_dot(attn, vh, ((1,), (0,))))
    attnout = jnp.concatenate(outs, axis=1)

    y = _dot(attnout, wout_ref[...], ((1,), (1,)))
    y_ref[0] = y + nullq_ref[...]

    lo = off_ref[b * (NB + 1) + i]
    hi = off_ref[b * (NB + 1) + i + 1]

    def add_row(j, carry):
        p = idx_ref[b * NUM_HEAVY_Q + j] - i * ROWS_PER_BLK
        y_ref[0, pl.ds(p, 1), :] += ro_ref[0, pl.ds(j, 1), :]
        return carry

    lax.fori_loop(lo, hi, add_row, 0)


def _light(x, ln_g, ln_b, wqkv, wout, nullq, offsets, idx_q, ro):
    grid = (B, NB)
    return pl.pallas_call(
        _k1_body,
        grid_spec=pltpu.PrefetchScalarGridSpec(
            num_scalar_prefetch=2,
            grid=grid,
            in_specs=[
                pl.BlockSpec((1, ROWS_PER_BLK, DIM), lambda b, i, *_: (b, i, 0)),
                pl.BlockSpec((1, WINDOW, DIM),
                             lambda b, i, *_: (b, jnp.maximum(i * WIN_PER_BLK - 1, 0), 0)),
                pl.BlockSpec((1, WINDOW, DIM),
                             lambda b, i, *_: (b, jnp.minimum(i * WIN_PER_BLK + WIN_PER_BLK,
                                                              NWIN - 1), 0)),
                pl.BlockSpec((1, DIM), lambda b, i, *_: (0, 0)),
                pl.BlockSpec((1, DIM), lambda b, i, *_: (0, 0)),
                pl.BlockSpec((3 * 512, DIM), lambda b, i, *_: (0, 0)),
                pl.BlockSpec((DIM, 512), lambda b, i, *_: (0, 0)),
                pl.BlockSpec((1, DIM), lambda b, i, *_: (0, 0)),
                pl.BlockSpec((1, NUM_HEAVY_Q, DIM), lambda b, i, *_: (b, 0, 0)),
            ],
            out_specs=pl.BlockSpec((1, ROWS_PER_BLK, DIM),
                                   lambda b, i, *_: (b, i, 0)),
        ),
        out_shape=jax.ShapeDtypeStruct((B, N, DIM), jnp.float32),
    )(offsets.reshape(-1), idx_q.reshape(-1),
      x, x, x, ln_g, ln_b, wqkv, wout, nullq, ro)


_SC_CH = 64


def _sc_gather(table, idx, nrows):
    info = plsc.get_sparse_core_info()
    nw = info.num_cores * info.num_subcores
    b_per_w = nrows // nw
    n_chunks = b_per_w // _SC_CH
    mesh = plsc.VectorSubcoreMesh(core_axis_name="c", subcore_axis_name="s")

    @functools.partial(
        pl.kernel, mesh=mesh,
        out_type=jax.ShapeDtypeStruct((nrows, DIM), jnp.float32),
        scratch_types=[
            pltpu.VMEM((_SC_CH,), jnp.int32),
            pltpu.VMEM((_SC_CH, DIM), jnp.float32),
            pltpu.SemaphoreType.DMA,
        ],
    )
    def k(table_hbm, idx_hbm, out_hbm, idx_v, rows_v, sem):
        wid = lax.axis_index("s") * info.num_cores + lax.axis_index("c")
        for c in range(n_chunks):
            base = wid * b_per_w + c * _SC_CH
            pltpu.sync_copy(idx_hbm.at[pl.ds(base, _SC_CH)], idx_v)
            pltpu.async_copy(table_hbm.at[idx_v], rows_v, sem).wait()
            pltpu.sync_copy(rows_v, out_hbm.at[pl.ds(base, _SC_CH)])

    return k(table, idx)


def _k2_body(s_ref, logk_ref, mask_ref):
    s = s_ref[...]
    logk = logk_ref[:, 0:1]

    def it(_, carry):
        a, bb = carry
        sb = (s + bb) / EPS
        m = jnp.max(sb, axis=-1, keepdims=True)
        lse = jnp.log(jnp.sum(jnp.exp(sb - m), axis=-1, keepdims=True)) + m
        a = EPS * (logk - lse)
        bb = -jnp.maximum(s + a, 0.0)
        return a, bb

    a0 = jnp.zeros_like(s[:, 0:1])
    a, bb = lax.fori_loop(0, N_ITERS, it, (a0, -s))
    mask_ref[...] = jnp.exp((s + a + bb) / EPS)


def _coor_descent(s4, logk4):
    return pl.pallas_call(
        _k2_body,
        out_shape=jax.ShapeDtypeStruct((4, N), jnp.float32),
    )(s4, logk4)


def _k3_body(rq_ref, rkv_ref, g_ref, qw_ref, kvw_ref, nkv_ref, outwt_ref,
             nullq_ref, ro_ref, xn_ref, cn_ref):
    h = pl.program_id(1)
    g = g_ref[0]

    def rms(t):
        n = jnp.sqrt(jnp.sum(t * t, axis=-1, keepdims=True))
        return t / jnp.maximum(n, 1e-12) * (DIM ** 0.5) * g

    @pl.when(h == 0)
    def _():
        xn_ref[...] = rms(rq_ref[0])
        cn_ref[...] = rms(rkv_ref[0])

    xn = xn_ref[...]
    cn = cn_ref[...]

    q = _dot(xn, qw_ref[...], ((1,), (1,)))
    kvh = _dot(cn, kvw_ref[...], ((1,), (1,)))
    k = kvh[:, :HEAVY_DH]
    v = kvh[:, HEAVY_DH:]
    nk = nkv_ref[0, 0]
    nv = nkv_ref[1, 0]

    scale = HEAVY_DH ** -0.5
    sim = _dot(q, k, ((1,), (1,))) * scale
    sim_null = _dot(q, nk, ((1,), (1,))) * scale
    m = jnp.maximum(jnp.max(sim, axis=-1, keepdims=True), sim_null)
    p = jnp.exp(sim - m)
    p_null = jnp.exp(sim_null - m)
    denom = jnp.sum(p, axis=-1, keepdims=True) + p_null
    o = (_dot(p, v, ((1,), (0,))) + p_null * nv) / denom

    contrib = _dot(o, outwt_ref[...], ((1,), (0,)))

    @pl.when(h == 0)
    def _():
        ro_ref[0] = contrib - nullq_ref[...]

    @pl.when(h > 0)
    def _():
        ro_ref[0] = ro_ref[0] + contrib


def _heavy(rq, rkv, g, q_w, kv_w, null_kv4, out_wt, nullq):
    grid = (B, HEAVY_HEADS)
    return pl.pallas_call(
        _k3_body,
        grid=grid,
        in_specs=[
            pl.BlockSpec((1, NUM_HEAVY_Q, DIM), lambda b, h: (b, 0, 0)),
            pl.BlockSpec((1, NUM_HEAVY_KV, DIM), lambda b, h: (b, 0, 0)),
            pl.BlockSpec((1, DIM), lambda b, h: (0, 0)),
            pl.BlockSpec((HEAVY_DH, DIM), lambda b, h: (h, 0)),
            pl.BlockSpec((2 * HEAVY_DH, DIM), lambda b, h: (h, 0)),
            pl.BlockSpec((2, 1, 1, HEAVY_DH), lambda b, h: (0, h, 0, 0)),
            pl.BlockSpec((HEAVY_DH, DIM), lambda b, h: (h, 0)),
            pl.BlockSpec((1, DIM), lambda b, h: (0, 0)),
        ],
        out_specs=pl.BlockSpec((1, NUM_HEAVY_Q, DIM), lambda b, h: (b, 0, 0)),
        out_shape=jax.ShapeDtypeStruct((B, NUM_HEAVY_Q, DIM), jnp.float32),
        scratch_shapes=[
            pltpu.VMEM((NUM_HEAVY_Q, DIM), jnp.float32),
            pltpu.VMEM((NUM_HEAVY_KV, DIM), jnp.float32),
        ],
        compiler_params=pltpu.CompilerParams(
            dimension_semantics=("arbitrary", "arbitrary")),
    )(rq, rkv, g, q_w, kv_w, null_kv4, out_wt, nullq)


def kernel(x, ln_g, ln_b, light_qkv_w, light_out_w, q_route_tok, kv_route_tok,
           heavy_norm_g, null_kv, heavy_q_w, heavy_kv_w, heavy_out_w,
           null_q_token):
    nullq = null_q_token.reshape(1, DIM)

    rtoks = jnp.concatenate([q_route_tok, kv_route_tok], axis=0)
    s2 = jnp.einsum('bnd,rd->brn', x, rtoks)
    s4 = jnp.concatenate([s2[:, 0], s2[:, 1]], axis=0)

    kq = jnp.float32(min(NUM_HEAVY_Q * FETCH_K_RATIO, float(N)))
    kkv = jnp.float32(min(NUM_HEAVY_KV * FETCH_K_RATIO, float(N)))
    logk4 = jnp.log(jnp.maximum(
        jnp.stack([kq, kq, kkv, kkv])[:, None], 1e-20))
    logk4 = jnp.broadcast_to(logk4, (4, 128))

    scores = _coor_descent(s4, logk4)
    _, idx_q = lax.top_k(scores[:B], NUM_HEAVY_Q)
    _, idx_kv = lax.top_k(scores[B:], NUM_HEAVY_KV)
    idx_q = jnp.sort(idx_q, axis=-1)
    idx_kv = jnp.sort(idx_kv, axis=-1)

    xf = x.reshape(B * N, DIM)
    boff = (jnp.arange(B, dtype=jnp.int32) * N)[:, None]
    rq = _sc_gather(xf, (idx_q + boff).reshape(-1),
                    B * NUM_HEAVY_Q).reshape(B, NUM_HEAVY_Q, DIM)
    rkv = _sc_gather(xf, (idx_kv + boff).reshape(-1),
                     B * NUM_HEAVY_KV).reshape(B, NUM_HEAVY_KV, DIM)

    null_kv4 = null_kv.reshape(2, HEAVY_HEADS, 1, HEAVY_DH)
    ro = _heavy(rq, rkv, heavy_norm_g.reshape(1, DIM), heavy_q_w, heavy_kv_w,
                null_kv4, heavy_out_w.T, nullq)

    bounds = jnp.arange(NB + 1, dtype=jnp.int32) * ROWS_PER_BLK
    offsets = jax.vmap(lambda r: jnp.searchsorted(r, bounds))(idx_q)
    offsets = offsets.astype(jnp.int32)

    return _light(x, ln_g.reshape(1, DIM), ln_b.reshape(1, DIM),
                  light_qkv_w, light_out_w, nullq, offsets, idx_q, ro)
